# Initial kernel scaffold; baseline (speedup 1.0000x reference)
import jax
import jax.numpy as jnp
from jax.experimental import pallas as pl


def kernel(recipe_id, ing, other_features, id_table, W_ing, b_ing, W_out, b_out):
    raise NotImplementedError("write your pallas kernel here")


# speedup vs baseline: 5.1975x; 5.1975x over previous
import functools

import jax
import jax.numpy as jnp
from jax import lax
from jax.experimental import pallas as pl
from jax.experimental.pallas import tpu as pltpu
from jax.experimental.pallas import tpu_sc as plsc

B = 4096
SEQ_LEN = 50
N_TOK = B * SEQ_LEN
ID_EMB = 32
ING_EMB = 32
ING_RAW = 64
OUT_SIZE = 64

CHUNK = 128
NW = 32
TOK_PER_W = N_TOK // NW
CHUNKS_PER_W = TOK_PER_W // CHUNK


def _gather_body(table_hbm, idx_hbm, out_hbm, idx_v, rows_v, sem):
    nc = 2
    wid = lax.axis_index("s") * nc + lax.axis_index("c")
    base = wid * TOK_PER_W

    pltpu.sync_copy(idx_hbm.at[wid], idx_v)

    def step(j, carry):
        pltpu.async_copy(table_hbm.at[idx_v.at[j]], rows_v, sem).wait()
        pltpu.sync_copy(rows_v, out_hbm.at[pl.ds(base + j * CHUNK, CHUNK)])
        return carry

    lax.fori_loop(0, CHUNKS_PER_W, step, 0)


@functools.partial(jax.jit, donate_argnums=())
def _sc_gather(id_table, idx_2d):
    mesh = plsc.VectorSubcoreMesh(core_axis_name="c", subcore_axis_name="s")
    fn = pl.kernel(
        _gather_body,
        mesh=mesh,
        out_type=jax.ShapeDtypeStruct((N_TOK, ID_EMB), jnp.float32),
        scratch_types=[
            pltpu.VMEM((CHUNKS_PER_W, CHUNK), jnp.int32),
            pltpu.VMEM((CHUNK, ID_EMB), jnp.float32),
            pltpu.SemaphoreType.DMA,
        ],
        compiler_params=pltpu.CompilerParams(use_tc_tiling_on_sc=False),
    )
    return fn(id_table, idx_2d)


def _dense_body(xid_ref, ing_ref, wing_ref, bing_ref, wout_ref, bout_ref, out_ref):
    xing = jnp.tanh(
        jnp.dot(ing_ref[...], wing_ref[...], preferred_element_type=jnp.float32)
        + bing_ref[...]
    )
    acc = jnp.dot(xid_ref[...], wout_ref[0:ID_EMB, :],
                  preferred_element_type=jnp.float32)
    acc = acc + jnp.dot(xing, wout_ref[ID_EMB:, :],
                        preferred_element_type=jnp.float32)
    out_ref[...] = jnp.tanh(acc + bout_ref[...])


TC_BLK = 2048


def _tc_dense(x_id, ing2d, W_ing, b_ing, W_out, b_out):
    grid = (N_TOK // TC_BLK,)
    return pl.pallas_call(
        _dense_body,
        grid=grid,
        in_specs=[
            pl.BlockSpec((TC_BLK, ID_EMB), lambda i: (i, 0)),
            pl.BlockSpec((TC_BLK, ING_RAW), lambda i: (i, 0)),
            pl.BlockSpec((ING_RAW, ING_EMB), lambda i: (0, 0)),
            pl.BlockSpec((1, ING_EMB), lambda i: (0, 0)),
            pl.BlockSpec((ID_EMB + ING_EMB, OUT_SIZE), lambda i: (0, 0)),
            pl.BlockSpec((1, OUT_SIZE), lambda i: (0, 0)),
        ],
        out_specs=pl.BlockSpec((TC_BLK, OUT_SIZE), lambda i: (i, 0)),
        out_shape=jax.ShapeDtypeStruct((N_TOK, OUT_SIZE), jnp.float32),
    )(x_id, ing2d, W_ing, b_ing, W_out, b_out)


def kernel(recipe_id, ing, other_features, id_table, W_ing, b_ing, W_out, b_out):
    idx_2d = recipe_id.astype(jnp.int32).reshape(NW, CHUNKS_PER_W, CHUNK)
    x_id = _sc_gather(id_table, idx_2d)
    ing2d = ing.reshape(N_TOK, ING_RAW)
    out = _tc_dense(x_id, ing2d, W_ing, b_ing.reshape(1, ING_EMB),
                    W_out, b_out.reshape(1, OUT_SIZE))
    return out.reshape(B, SEQ_LEN, OUT_SIZE)


# speedup vs baseline: 6.5039x; 1.2513x over previous
import functools

import jax
import jax.numpy as jnp
from jax import lax
from jax.experimental import pallas as pl
from jax.experimental.pallas import tpu as pltpu
from jax.experimental.pallas import tpu_sc as plsc

B = 4096
NUM_IDS = 1000000
SEQ_LEN = 50
N_TOK = B * SEQ_LEN
ID_EMB = 32
ING_EMB = 32
ING_RAW = 64
OUT_SIZE = 64

CHUNK = 128
NW = 32
TOK_PER_W = N_TOK // NW
CPW = TOK_PER_W // CHUNK
CHUNKS_PER_L = B // CHUNK


def _gather_body(table_hbm, idx_hbm, out_hbm, idx_v, rows_v, sem):
    nc = 2
    wid = lax.axis_index("s") * nc + lax.axis_index("c")

    pltpu.sync_copy(idx_hbm.at[wid], idx_v)

    def chunk_step(j, carry):
        pltpu.async_copy(table_hbm.at[idx_v.at[j]], rows_v, sem).wait()
        c = wid * CPW + j
        l = lax.shift_right_logical(c, 5)
        b0 = pl.multiple_of(
            lax.shift_left(lax.bitwise_and(c, CHUNKS_PER_L - 1), 7), CHUNK)
        pltpu.sync_copy(rows_v, out_hbm.at[l, pl.ds(b0, CHUNK), :])
        return carry

    lax.fori_loop(0, CPW, chunk_step, 0)


@jax.jit
def _sc_gather(table_p, idx_3d):
    mesh = plsc.VectorSubcoreMesh(core_axis_name="c", subcore_axis_name="s")
    fn = pl.kernel(
        _gather_body,
        mesh=mesh,
        out_type=jax.ShapeDtypeStruct((SEQ_LEN, B, ID_EMB), jnp.float32),
        scratch_types=[
            pltpu.VMEM((CPW, CHUNK), jnp.int32),
            pltpu.VMEM((CHUNK, ID_EMB), jnp.float32),
            pltpu.SemaphoreType.DMA,
        ],
        compiler_params=pltpu.CompilerParams(use_tc_tiling_on_sc=False),
    )
    return fn(table_p, idx_3d)


def _dense_body(xid_ref, ingt_ref, wingT_ref, bing_ref, w1T_ref, w2T_ref,
                bout_ref, out_ref):
    xing = jnp.tanh(
        jnp.dot(wingT_ref[...], ingt_ref[0], preferred_element_type=jnp.float32)
        + bing_ref[...]
    )
    acc = lax.dot_general(
        w1T_ref[...], xid_ref[0], (((1,), (1,)), ((), ())),
        preferred_element_type=jnp.float32)
    acc = acc + jnp.dot(w2T_ref[...], xing, preferred_element_type=jnp.float32)
    out_ref[0] = jnp.tanh(acc + bout_ref[...])


B2 = 1024


def _tc_dense(x_id_lm, ing_t, wingT, bing, w1T, w2T, bout):
    grid = (SEQ_LEN, B // B2)
    return pl.pallas_call(
        _dense_body,
        grid=grid,
        in_specs=[
            pl.BlockSpec((1, B2, ID_EMB), lambda l, b: (l, b, 0)),
            pl.BlockSpec((1, ING_RAW, B2), lambda l, b: (l, 0, b)),
            pl.BlockSpec((ING_EMB, ING_RAW), lambda l, b: (0, 0)),
            pl.BlockSpec((ING_EMB, 1), lambda l, b: (0, 0)),
            pl.BlockSpec((OUT_SIZE, ID_EMB), lambda l, b: (0, 0)),
            pl.BlockSpec((OUT_SIZE, ING_EMB), lambda l, b: (0, 0)),
            pl.BlockSpec((OUT_SIZE, 1), lambda l, b: (0, 0)),
        ],
        out_specs=pl.BlockSpec((1, OUT_SIZE, B2), lambda l, b: (l, 0, b)),
        out_shape=jax.ShapeDtypeStruct((SEQ_LEN, OUT_SIZE, B), jnp.float32),
    )(x_id_lm, ing_t, wingT, bing, w1T, w2T, bout)


def kernel(recipe_id, ing, other_features, id_table, W_ing, b_ing, W_out, b_out):
    idx_3d = (recipe_id.astype(jnp.int32).transpose(1, 0)
              .reshape(NW, CPW, CHUNK))
    x_id_lm = _sc_gather(id_table, idx_3d)
    ing_t = jnp.transpose(ing, (1, 2, 0))
    out_t = _tc_dense(
        x_id_lm, ing_t,
        W_ing.T,
        b_ing.reshape(ING_EMB, 1),
        W_out[:ID_EMB].T,
        W_out[ID_EMB:].T,
        b_out.reshape(OUT_SIZE, 1),
    )
    return jnp.transpose(out_t, (2, 0, 1))
